# Initial kernel scaffold; baseline (speedup 1.0000x reference)
import jax
import jax.numpy as jnp
from jax.experimental import pallas as pl


def kernel(x, conv_w, conv_b, gamma, beta):
    raise NotImplementedError("write your pallas kernel here")


# speedup vs baseline: 1.0418x; 1.0418x over previous
import functools

import jax
import jax.numpy as jnp
from jax.experimental import pallas as pl
from jax.experimental.pallas import tpu as pltpu

_LANES = 128


def _conv_pool_kernel(xw_ref, w_ref, b_ref, pooled_ref, psum_ref, psq_ref,
                      *, n_phase, cout):
    m = jnp.dot(w_ref[...], xw_ref[...],
                preferred_element_type=jnp.float32)
    m = jnp.max(m.reshape(n_phase, cout, m.shape[-1]), axis=0)
    m = jnp.maximum(m + b_ref[...], 0.0)

    pooled_ref[...] = m
    s = jnp.sum(m, axis=1, keepdims=True)
    sq = jnp.sum(m * m, axis=1, keepdims=True)
    psum_ref[...] = jnp.broadcast_to(s, psum_ref.shape)
    psq_ref[...] = jnp.broadcast_to(sq, psq_ref.shape)


def _bn_kernel(pooled_ref, psum_ref, psq_ref, gamma_ref, beta_ref, o_ref,
               *, inv_count, eps):
    inv_rep = 1.0 / float(_LANES)
    s = jnp.sum(psum_ref[...], axis=1, keepdims=True) * inv_rep
    sq = jnp.sum(psq_ref[...], axis=1, keepdims=True) * inv_rep
    mean = s * inv_count
    var = jnp.maximum(sq * inv_count - mean * mean, 0.0)
    scale = jax.lax.rsqrt(var + eps) * gamma_ref[...]
    shift = beta_ref[...] - mean * scale
    o_ref[...] = pooled_ref[...] * scale + shift


def kernel(x, conv_w, conv_b, gamma, beta):
    eps = 1e-5
    B, Cin, D, H, W = x.shape
    Cout = conv_w.shape[0]
    Do, Ho, Wo = D // 2, H // 2, W // 2
    Nsp = Do * Ho * Wo
    K = 64 * Cin
    f32 = jnp.float32
    bf16 = jnp.bfloat16

    xb = x.astype(bf16)
    xp = jnp.pad(xb, ((0, 0), (0, 0), (1, 1), (1, 1), (1, 1)))
    slabs = []
    for dw in range(4):
        for hw in range(4):
            for ww in range(4):
                slabs.append(xp[:, :, dw:dw + D:2, hw:hw + H:2, ww:ww + W:2])
    xw = jnp.stack(slabs, axis=2).reshape(B, K, Nsp)

    w = conv_w.astype(f32)
    phase_w = []
    for pd in range(2):
        for ph in range(2):
            for pw in range(2):
                wp = jnp.zeros((Cout, Cin, 4, 4, 4), f32)
                wp = wp.at[:, :, pd:pd + 3, ph:ph + 3, pw:pw + 3].set(w)
                phase_w.append(wp.reshape(Cout, K))
    w_all = jnp.concatenate(phase_w, axis=0).astype(bf16)
    bias = conv_b.astype(f32).reshape(Cout, 1)

    conv_body = functools.partial(_conv_pool_kernel, n_phase=8, cout=Cout)
    pooled, psum, psq = pl.pallas_call(
        conv_body,
        out_shape=(
            jax.ShapeDtypeStruct((B, Cout, Nsp), f32),
            jax.ShapeDtypeStruct((Cout, B * _LANES), f32),
            jax.ShapeDtypeStruct((Cout, B * _LANES), f32),
        ),
        grid=(B,),
        in_specs=[
            pl.BlockSpec((None, K, Nsp), lambda b: (b, 0, 0)),
            pl.BlockSpec((8 * Cout, K), lambda b: (0, 0)),
            pl.BlockSpec((Cout, 1), lambda b: (0, 0)),
        ],
        out_specs=(
            pl.BlockSpec((None, Cout, Nsp), lambda b: (b, 0, 0)),
            pl.BlockSpec((Cout, _LANES), lambda b: (0, b)),
            pl.BlockSpec((Cout, _LANES), lambda b: (0, b)),
        ),
        compiler_params=pltpu.CompilerParams(
            dimension_semantics=("parallel",)),
    )(xw, w_all, bias)

    bn_body = functools.partial(_bn_kernel,
                                inv_count=1.0 / float(B * Nsp), eps=float(eps))
    out_flat = pl.pallas_call(
        bn_body,
        out_shape=jax.ShapeDtypeStruct((B, Cout, Nsp), f32),
        grid=(B,),
        in_specs=[
            pl.BlockSpec((None, Cout, Nsp), lambda b: (b, 0, 0)),
            pl.BlockSpec((Cout, B * _LANES), lambda b: (0, 0)),
            pl.BlockSpec((Cout, B * _LANES), lambda b: (0, 0)),
            pl.BlockSpec((Cout, 1), lambda b: (0, 0)),
            pl.BlockSpec((Cout, 1), lambda b: (0, 0)),
        ],
        out_specs=pl.BlockSpec((None, Cout, Nsp), lambda b: (b, 0, 0)),
        compiler_params=pltpu.CompilerParams(
            dimension_semantics=("parallel",)),
    )(pooled, psum, psq,
      gamma.astype(f32).reshape(Cout, 1), beta.astype(f32).reshape(Cout, 1))

    return out_flat.reshape(B, Cout, Do, Ho, Wo)


# speedup vs baseline: 71.5683x; 68.6960x over previous
import functools

import jax
import jax.numpy as jnp
from jax.experimental import pallas as pl
from jax.experimental.pallas import tpu as pltpu

_LANES = 128


def _conv_pool_kernel(xw_ref, w_ref, b_ref, pooled_ref, psum_ref, psq_ref,
                      *, n_phase, cout):
    m = jnp.dot(w_ref[...], xw_ref[...],
                preferred_element_type=jnp.float32)
    m = jnp.max(m.reshape(n_phase, cout, m.shape[-1]), axis=0)
    m = jnp.maximum(m + b_ref[...], 0.0)

    pooled_ref[...] = m
    s = jnp.sum(m, axis=1, keepdims=True)
    sq = jnp.sum(m * m, axis=1, keepdims=True)
    psum_ref[...] = jnp.broadcast_to(s, psum_ref.shape)
    psq_ref[...] = jnp.broadcast_to(sq, psq_ref.shape)


def _bn_kernel(pooled_ref, psum_ref, psq_ref, gamma_ref, beta_ref, o_ref,
               *, inv_count, eps):
    inv_rep = 1.0 / float(_LANES)
    s = jnp.sum(psum_ref[...], axis=1, keepdims=True) * inv_rep
    sq = jnp.sum(psq_ref[...], axis=1, keepdims=True) * inv_rep
    mean = s * inv_count
    var = jnp.maximum(sq * inv_count - mean * mean, 0.0)
    scale = jax.lax.rsqrt(var + eps) * gamma_ref[...]
    shift = beta_ref[...] - mean * scale
    o_ref[...] = pooled_ref[...] * scale + shift


def kernel(x, conv_w, conv_b, gamma, beta):
    eps = 1e-5
    B, Cin, D, H, W = x.shape
    Cout = conv_w.shape[0]
    Do, Ho, Wo = D // 2, H // 2, W // 2
    Nsp = Do * Ho * Wo
    K = 64 * Cin
    f32 = jnp.float32
    bf16 = jnp.bfloat16

    xb = x.astype(bf16)
    xw = jax.lax.conv_general_dilated_patches(
        xb, filter_shape=(4, 4, 4), window_strides=(2, 2, 2),
        padding=((1, 1), (1, 1), (1, 1)))
    xw = xw.reshape(B, K, Nsp)

    w = conv_w.astype(f32)
    phase_w = []
    for pd in range(2):
        for ph in range(2):
            for pw in range(2):
                wp = jnp.zeros((Cout, Cin, 4, 4, 4), f32)
                wp = wp.at[:, :, pd:pd + 3, ph:ph + 3, pw:pw + 3].set(w)
                phase_w.append(wp.reshape(Cout, K))
    w_all = jnp.concatenate(phase_w, axis=0).astype(bf16)
    bias = conv_b.astype(f32).reshape(Cout, 1)

    conv_body = functools.partial(_conv_pool_kernel, n_phase=8, cout=Cout)
    pooled, psum, psq = pl.pallas_call(
        conv_body,
        out_shape=(
            jax.ShapeDtypeStruct((B, Cout, Nsp), f32),
            jax.ShapeDtypeStruct((Cout, B * _LANES), f32),
            jax.ShapeDtypeStruct((Cout, B * _LANES), f32),
        ),
        grid=(B,),
        in_specs=[
            pl.BlockSpec((None, K, Nsp), lambda b: (b, 0, 0)),
            pl.BlockSpec((8 * Cout, K), lambda b: (0, 0)),
            pl.BlockSpec((Cout, 1), lambda b: (0, 0)),
        ],
        out_specs=(
            pl.BlockSpec((None, Cout, Nsp), lambda b: (b, 0, 0)),
            pl.BlockSpec((Cout, _LANES), lambda b: (0, b)),
            pl.BlockSpec((Cout, _LANES), lambda b: (0, b)),
        ),
        compiler_params=pltpu.CompilerParams(
            dimension_semantics=("parallel",)),
    )(xw, w_all, bias)

    bn_body = functools.partial(_bn_kernel,
                                inv_count=1.0 / float(B * Nsp), eps=float(eps))
    out_flat = pl.pallas_call(
        bn_body,
        out_shape=jax.ShapeDtypeStruct((B, Cout, Nsp), f32),
        grid=(B,),
        in_specs=[
            pl.BlockSpec((None, Cout, Nsp), lambda b: (b, 0, 0)),
            pl.BlockSpec((Cout, B * _LANES), lambda b: (0, 0)),
            pl.BlockSpec((Cout, B * _LANES), lambda b: (0, 0)),
            pl.BlockSpec((Cout, 1), lambda b: (0, 0)),
            pl.BlockSpec((Cout, 1), lambda b: (0, 0)),
        ],
        out_specs=pl.BlockSpec((None, Cout, Nsp), lambda b: (b, 0, 0)),
        compiler_params=pltpu.CompilerParams(
            dimension_semantics=("parallel",)),
    )(pooled, psum, psq,
      gamma.astype(f32).reshape(Cout, 1), beta.astype(f32).reshape(Cout, 1))

    return out_flat.reshape(B, Cout, Do, Ho, Wo)


# speedup vs baseline: 75.3929x; 1.0534x over previous
import functools

import jax
import jax.numpy as jnp
import numpy as np
from jax.experimental import pallas as pl
from jax.experimental.pallas import tpu as pltpu

_LANES = 128


def _roll_lanes(v, k):
    if k == 0:
        return v
    return jnp.concatenate([v[:, k:], v[:, :k]], axis=1)


def _conv_pool_kernel(x_ref, w1_ref, b_ref, sc_ref,
                      pooled_ref, psum_ref, psq_ref, *, dims):
    cin, cout, hp, wp = dims
    hw = hp * wp

    x = x_ref[...]

    slabs = []
    for kh in range(3):
        for kw in range(3):
            slabs.append(_roll_lanes(x, (kh - 1) * wp + (kw - 1)))
    u = jnp.concatenate(slabs, axis=0)

    t = jnp.dot(w1_ref[...], u,
                preferred_element_type=jnp.float32)

    y = (_roll_lanes(t[:cout], -hw)
         + t[cout:2 * cout]
         + _roll_lanes(t[2 * cout:], hw))

    m = jnp.maximum(y + b_ref[...], 0.0)
    m = jnp.maximum(m, _roll_lanes(m, 1))
    m = jnp.maximum(m, _roll_lanes(m, wp))
    m = jnp.maximum(m, _roll_lanes(m, hw))

    pooled = jnp.dot(m.astype(jnp.bfloat16), sc_ref[...],
                     preferred_element_type=jnp.float32)
    pooled_ref[...] = pooled

    s = jnp.sum(pooled, axis=1, keepdims=True)
    sq = jnp.sum(pooled * pooled, axis=1, keepdims=True)
    psum_ref[...] = jnp.broadcast_to(s, psum_ref.shape)
    psq_ref[...] = jnp.broadcast_to(sq, psq_ref.shape)


def _bn_kernel(pooled_ref, psum_ref, psq_ref, gamma_ref, beta_ref, o_ref,
               *, inv_count, eps):
    inv_rep = 1.0 / float(_LANES)
    s = jnp.sum(psum_ref[...], axis=1, keepdims=True) * inv_rep
    sq = jnp.sum(psq_ref[...], axis=1, keepdims=True) * inv_rep
    mean = s * inv_count
    var = jnp.maximum(sq * inv_count - mean * mean, 0.0)
    scale = jax.lax.rsqrt(var + eps) * gamma_ref[...]
    shift = beta_ref[...] - mean * scale
    o_ref[...] = pooled_ref[...] * scale + shift


def kernel(x, conv_w, conv_b, gamma, beta):
    eps = 1e-5
    B, Cin, D, H, W = x.shape
    Cout = conv_w.shape[0]
    Do, Ho, Wo = D // 2, H // 2, W // 2
    Nsp = Do * Ho * Wo
    Dp, Hp, Wp = D + 2, H + 2, W + 2
    L = Dp * Hp * Wp
    f32 = jnp.float32
    bf16 = jnp.bfloat16

    xp = jnp.pad(x.astype(bf16), ((0, 0), (0, 0), (1, 1), (1, 1), (1, 1)))
    x_flat = xp.reshape(B, Cin, L)

    w1 = (conv_w.astype(f32).transpose(2, 0, 3, 4, 1)
          .reshape(3 * Cout, 9 * Cin).astype(bf16))
    bias = conv_b.astype(f32).reshape(Cout, 1)

    sel = np.zeros((L, Nsp), np.float32)
    for od in range(Do):
        for oh in range(Ho):
            for ow in range(Wo):
                l = (2 * od + 1) * Hp * Wp + (2 * oh + 1) * Wp + (2 * ow + 1)
                sel[l, (od * Ho + oh) * Wo + ow] = 1.0
    sc = jnp.asarray(sel, bf16)

    conv_body = functools.partial(_conv_pool_kernel,
                                  dims=(Cin, Cout, Hp, Wp))
    pooled, psum, psq = pl.pallas_call(
        conv_body,
        out_shape=(
            jax.ShapeDtypeStruct((B, Cout, Nsp), f32),
            jax.ShapeDtypeStruct((Cout, B * _LANES), f32),
            jax.ShapeDtypeStruct((Cout, B * _LANES), f32),
        ),
        grid=(B,),
        in_specs=[
            pl.BlockSpec((None, Cin, L), lambda b: (b, 0, 0)),
            pl.BlockSpec((3 * Cout, 9 * Cin), lambda b: (0, 0)),
            pl.BlockSpec((Cout, 1), lambda b: (0, 0)),
            pl.BlockSpec((L, Nsp), lambda b: (0, 0)),
        ],
        out_specs=(
            pl.BlockSpec((None, Cout, Nsp), lambda b: (b, 0, 0)),
            pl.BlockSpec((Cout, _LANES), lambda b: (0, b)),
            pl.BlockSpec((Cout, _LANES), lambda b: (0, b)),
        ),
        compiler_params=pltpu.CompilerParams(
            dimension_semantics=("parallel",)),
    )(x_flat, w1, bias, sc)

    bn_body = functools.partial(_bn_kernel,
                                inv_count=1.0 / float(B * Nsp), eps=float(eps))
    out_flat = pl.pallas_call(
        bn_body,
        out_shape=jax.ShapeDtypeStruct((B, Cout, Nsp), f32),
        grid=(B,),
        in_specs=[
            pl.BlockSpec((None, Cout, Nsp), lambda b: (b, 0, 0)),
            pl.BlockSpec((Cout, B * _LANES), lambda b: (0, 0)),
            pl.BlockSpec((Cout, B * _LANES), lambda b: (0, 0)),
            pl.BlockSpec((Cout, 1), lambda b: (0, 0)),
            pl.BlockSpec((Cout, 1), lambda b: (0, 0)),
        ],
        out_specs=pl.BlockSpec((None, Cout, Nsp), lambda b: (b, 0, 0)),
        compiler_params=pltpu.CompilerParams(
            dimension_semantics=("parallel",)),
    )(pooled, psum, psq,
      gamma.astype(f32).reshape(Cout, 1), beta.astype(f32).reshape(Cout, 1))

    return out_flat.reshape(B, Cout, Do, Ho, Wo)


# speedup vs baseline: 210.2957x; 2.7893x over previous
import functools

import jax
import jax.numpy as jnp
import numpy as np
from jax.experimental import pallas as pl
from jax.experimental.pallas import tpu as pltpu

_LANES = 128


def _roll_lanes(v, k):
    if k == 0:
        return v
    return jnp.concatenate([v[:, k:], v[:, :k]], axis=1)


def _conv_pool_kernel(x_ref, w1_ref, b_ref, hwm_ref, dm_ref, sc_ref,
                      pooled_ref, psum_ref, psq_ref, *, dims):
    g, cin, cout, h, w = dims
    hw = h * w
    bf16 = jnp.bfloat16

    mrows = []
    for e in range(g):
        x = x_ref[e]

        slabs = []
        for kh in range(3):
            for kw in range(3):
                j = kh * 3 + kw
                sh = (kh - 1) * w + (kw - 1)
                slabs.append(_roll_lanes(x, sh) * hwm_ref[j:j + 1, :])
        u = jnp.concatenate(slabs, axis=0)

        t = jnp.dot(w1_ref[...], u,
                    preferred_element_type=jnp.float32)

        y = (_roll_lanes(t[:cout], -hw) * dm_ref[0:1, :]
             + t[cout:2 * cout]
             + _roll_lanes(t[2 * cout:], hw) * dm_ref[1:2, :])

        m = jnp.maximum(y + b_ref[...], 0.0)
        m = jnp.maximum(m, _roll_lanes(m, 1))
        m = jnp.maximum(m, _roll_lanes(m, w))
        m = jnp.maximum(m, _roll_lanes(m, hw))

        mrows.append(m.astype(bf16))

    mall = jnp.concatenate(mrows, axis=0)
    pall = jnp.dot(mall, sc_ref[...],
                   preferred_element_type=jnp.float32)

    psum = None
    psq = None
    for e in range(g):
        pc = pall[e * cout:(e + 1) * cout]
        pooled_ref[e] = pc
        s = jnp.sum(pc, axis=1, keepdims=True)
        sq = jnp.sum(pc * pc, axis=1, keepdims=True)
        psum = s if psum is None else psum + s
        psq = sq if psq is None else psq + sq

    psum_ref[...] = jnp.broadcast_to(psum, psum_ref.shape)
    psq_ref[...] = jnp.broadcast_to(psq, psq_ref.shape)


def _bn_kernel(pooled_ref, psum_ref, psq_ref, gamma_ref, beta_ref, o_ref,
               *, inv_count, eps):
    inv_rep = 1.0 / float(_LANES)
    s = jnp.sum(psum_ref[...], axis=1, keepdims=True) * inv_rep
    sq = jnp.sum(psq_ref[...], axis=1, keepdims=True) * inv_rep
    mean = s * inv_count
    var = jnp.maximum(sq * inv_count - mean * mean, 0.0)
    scale = jax.lax.rsqrt(var + eps) * gamma_ref[...]
    shift = beta_ref[...] - mean * scale
    o_ref[...] = pooled_ref[...] * scale + shift


def kernel(x, conv_w, conv_b, gamma, beta):
    eps = 1e-5
    B, Cin, D, H, W = x.shape
    Cout = conv_w.shape[0]
    Do, Ho, Wo = D // 2, H // 2, W // 2
    Nsp = Do * Ho * Wo
    S = D * H * W
    f32 = jnp.float32
    bf16 = jnp.bfloat16

    G = next(g for g in (8, 4, 2, 1) if B % g == 0)
    NT = B // G

    x_flat = x.astype(bf16).reshape(B, Cin, S)

    w1 = (conv_w.astype(f32).transpose(2, 0, 3, 4, 1)
          .reshape(3 * Cout, 9 * Cin).astype(bf16))
    bias = conv_b.astype(f32).reshape(Cout, 1)

    li = np.arange(S)
    hh = (li // W) % H
    ww = li % W
    dd = li // (H * W)
    hwm = np.ones((9, S), np.float32)
    for kh in range(3):
        for kw in range(3):
            bad = np.zeros(S, bool)
            if kh == 0:
                bad |= hh == 0
            if kh == 2:
                bad |= hh == H - 1
            if kw == 0:
                bad |= ww == 0
            if kw == 2:
                bad |= ww == W - 1
            hwm[kh * 3 + kw, bad] = 0.0
    dm = np.ones((2, S), np.float32)
    dm[0, dd == 0] = 0.0
    dm[1, dd == D - 1] = 0.0

    sel = np.zeros((S, Nsp), np.float32)
    for od in range(Do):
        for oh in range(Ho):
            for ow in range(Wo):
                k = (2 * od * H + 2 * oh) * W + 2 * ow
                sel[k, (od * Ho + oh) * Wo + ow] = 1.0

    hwm_j = jnp.asarray(hwm, bf16)
    dm_j = jnp.asarray(dm, f32)
    sc_j = jnp.asarray(sel, bf16)

    conv_body = functools.partial(_conv_pool_kernel,
                                  dims=(G, Cin, Cout, H, W))
    pooled, psum, psq = pl.pallas_call(
        conv_body,
        out_shape=(
            jax.ShapeDtypeStruct((B, Cout, Nsp), f32),
            jax.ShapeDtypeStruct((Cout, NT * _LANES), f32),
            jax.ShapeDtypeStruct((Cout, NT * _LANES), f32),
        ),
        grid=(NT,),
        in_specs=[
            pl.BlockSpec((G, Cin, S), lambda i: (i, 0, 0)),
            pl.BlockSpec((3 * Cout, 9 * Cin), lambda i: (0, 0)),
            pl.BlockSpec((Cout, 1), lambda i: (0, 0)),
            pl.BlockSpec((9, S), lambda i: (0, 0)),
            pl.BlockSpec((2, S), lambda i: (0, 0)),
            pl.BlockSpec((S, Nsp), lambda i: (0, 0)),
        ],
        out_specs=(
            pl.BlockSpec((G, Cout, Nsp), lambda i: (i, 0, 0)),
            pl.BlockSpec((Cout, _LANES), lambda i: (0, i)),
            pl.BlockSpec((Cout, _LANES), lambda i: (0, i)),
        ),
        compiler_params=pltpu.CompilerParams(
            dimension_semantics=("parallel",)),
    )(x_flat, w1, bias, hwm_j, dm_j, sc_j)

    bn_body = functools.partial(_bn_kernel,
                                inv_count=1.0 / float(B * Nsp), eps=float(eps))
    out_flat = pl.pallas_call(
        bn_body,
        out_shape=jax.ShapeDtypeStruct((B, Cout, Nsp), f32),
        grid=(NT,),
        in_specs=[
            pl.BlockSpec((G, Cout, Nsp), lambda i: (i, 0, 0)),
            pl.BlockSpec((Cout, NT * _LANES), lambda i: (0, 0)),
            pl.BlockSpec((Cout, NT * _LANES), lambda i: (0, 0)),
            pl.BlockSpec((Cout, 1), lambda i: (0, 0)),
            pl.BlockSpec((Cout, 1), lambda i: (0, 0)),
        ],
        out_specs=pl.BlockSpec((G, Cout, Nsp), lambda i: (i, 0, 0)),
        compiler_params=pltpu.CompilerParams(
            dimension_semantics=("parallel",)),
    )(pooled, psum, psq,
      gamma.astype(f32).reshape(Cout, 1), beta.astype(f32).reshape(Cout, 1))

    return out_flat.reshape(B, Cout, Do, Ho, Wo)


# speedup vs baseline: 224.0851x; 1.0656x over previous
import functools

import jax
import jax.numpy as jnp
import numpy as np
from jax.experimental import pallas as pl
from jax.experimental.pallas import tpu as pltpu

_LANES = 128


def _roll_lanes(v, k):
    if k == 0:
        return v
    return jnp.concatenate([v[:, k:], v[:, :k]], axis=1)


def _conv_pool_kernel(x_ref, w1_ref, b_ref, hwm_ref, dm_ref, sc_ref,
                      pooled_ref, psum_ref, psq_ref, *, dims):
    g, cin, cout, h, w = dims
    hw = h * w
    bf16 = jnp.bfloat16

    mrows = []
    for e in range(g):
        x = x_ref[e].astype(bf16)

        slabs = []
        for kh in range(3):
            for kw in range(3):
                j = kh * 3 + kw
                sh = (kh - 1) * w + (kw - 1)
                slabs.append(_roll_lanes(x, sh) * hwm_ref[j:j + 1, :])
        u = jnp.concatenate(slabs, axis=0)

        t = jnp.dot(w1_ref[...], u,
                    preferred_element_type=jnp.float32)

        y = (_roll_lanes(t[:cout], -hw) * dm_ref[0:1, :]
             + t[cout:2 * cout]
             + _roll_lanes(t[2 * cout:], hw) * dm_ref[1:2, :])

        m = jnp.maximum(y + b_ref[...], 0.0)
        m = jnp.maximum(m, _roll_lanes(m, 1))
        m = jnp.maximum(m, _roll_lanes(m, w))
        m = jnp.maximum(m, _roll_lanes(m, hw))

        mrows.append(m.astype(bf16))

    mall = jnp.concatenate(mrows, axis=0)
    pall = jnp.dot(mall, sc_ref[...],
                   preferred_element_type=jnp.float32)

    psum = None
    psq = None
    for e in range(g):
        pc = pall[e * cout:(e + 1) * cout]
        pooled_ref[e] = pc
        s = jnp.sum(pc, axis=1, keepdims=True)
        sq = jnp.sum(pc * pc, axis=1, keepdims=True)
        psum = s if psum is None else psum + s
        psq = sq if psq is None else psq + sq

    psum_ref[...] = jnp.broadcast_to(psum, psum_ref.shape)
    psq_ref[...] = jnp.broadcast_to(psq, psq_ref.shape)


def _bn_kernel(pooled_ref, psum_ref, psq_ref, gamma_ref, beta_ref, o_ref,
               *, inv_count, eps):
    inv_rep = 1.0 / float(_LANES)
    s = jnp.sum(psum_ref[...], axis=1, keepdims=True) * inv_rep
    sq = jnp.sum(psq_ref[...], axis=1, keepdims=True) * inv_rep
    mean = s * inv_count
    var = jnp.maximum(sq * inv_count - mean * mean, 0.0)
    scale = jax.lax.rsqrt(var + eps) * gamma_ref[...]
    shift = beta_ref[...] - mean * scale
    o_ref[...] = pooled_ref[...] * scale + shift


def kernel(x, conv_w, conv_b, gamma, beta):
    eps = 1e-5
    B, Cin, D, H, W = x.shape
    Cout = conv_w.shape[0]
    Do, Ho, Wo = D // 2, H // 2, W // 2
    Nsp = Do * Ho * Wo
    S = D * H * W
    f32 = jnp.float32
    bf16 = jnp.bfloat16

    G = next(g for g in (12, 8, 6, 4, 2, 1) if B % g == 0)
    NT = B // G

    x_flat = x.reshape(B, Cin, S)

    w1 = (conv_w.astype(f32).transpose(2, 0, 3, 4, 1)
          .reshape(3 * Cout, 9 * Cin).astype(bf16))
    bias = conv_b.astype(f32).reshape(Cout, 1)

    li = np.arange(S)
    hh = (li // W) % H
    ww = li % W
    dd = li // (H * W)
    hwm = np.ones((9, S), np.float32)
    for kh in range(3):
        for kw in range(3):
            bad = np.zeros(S, bool)
            if kh == 0:
                bad |= hh == 0
            if kh == 2:
                bad |= hh == H - 1
            if kw == 0:
                bad |= ww == 0
            if kw == 2:
                bad |= ww == W - 1
            hwm[kh * 3 + kw, bad] = 0.0
    dm = np.ones((2, S), np.float32)
    dm[0, dd == 0] = 0.0
    dm[1, dd == D - 1] = 0.0

    sel = np.zeros((S, Nsp), np.float32)
    for od in range(Do):
        for oh in range(Ho):
            for ow in range(Wo):
                k = (2 * od * H + 2 * oh) * W + 2 * ow
                sel[k, (od * Ho + oh) * Wo + ow] = 1.0

    hwm_j = jnp.asarray(hwm, bf16)
    dm_j = jnp.asarray(dm, f32)
    sc_j = jnp.asarray(sel, bf16)

    conv_body = functools.partial(_conv_pool_kernel,
                                  dims=(G, Cin, Cout, H, W))
    pooled, psum, psq = pl.pallas_call(
        conv_body,
        out_shape=(
            jax.ShapeDtypeStruct((B, Cout, Nsp), f32),
            jax.ShapeDtypeStruct((Cout, NT * _LANES), f32),
            jax.ShapeDtypeStruct((Cout, NT * _LANES), f32),
        ),
        grid=(NT,),
        in_specs=[
            pl.BlockSpec((G, Cin, S), lambda i: (i, 0, 0)),
            pl.BlockSpec((3 * Cout, 9 * Cin), lambda i: (0, 0)),
            pl.BlockSpec((Cout, 1), lambda i: (0, 0)),
            pl.BlockSpec((9, S), lambda i: (0, 0)),
            pl.BlockSpec((2, S), lambda i: (0, 0)),
            pl.BlockSpec((S, Nsp), lambda i: (0, 0)),
        ],
        out_specs=(
            pl.BlockSpec((G, Cout, Nsp), lambda i: (i, 0, 0)),
            pl.BlockSpec((Cout, _LANES), lambda i: (0, i)),
            pl.BlockSpec((Cout, _LANES), lambda i: (0, i)),
        ),
        compiler_params=pltpu.CompilerParams(
            dimension_semantics=("parallel",)),
    )(x_flat, w1, bias, hwm_j, dm_j, sc_j)

    bn_body = functools.partial(_bn_kernel,
                                inv_count=1.0 / float(B * Nsp), eps=float(eps))
    out_flat = pl.pallas_call(
        bn_body,
        out_shape=jax.ShapeDtypeStruct((B, Cout, Nsp), f32),
        grid=(NT,),
        in_specs=[
            pl.BlockSpec((G, Cout, Nsp), lambda i: (i, 0, 0)),
            pl.BlockSpec((Cout, NT * _LANES), lambda i: (0, 0)),
            pl.BlockSpec((Cout, NT * _LANES), lambda i: (0, 0)),
            pl.BlockSpec((Cout, 1), lambda i: (0, 0)),
            pl.BlockSpec((Cout, 1), lambda i: (0, 0)),
        ],
        out_specs=pl.BlockSpec((G, Cout, Nsp), lambda i: (i, 0, 0)),
        compiler_params=pltpu.CompilerParams(
            dimension_semantics=("parallel",)),
    )(pooled, psum, psq,
      gamma.astype(f32).reshape(Cout, 1), beta.astype(f32).reshape(Cout, 1))

    return out_flat.reshape(B, Cout, Do, Ho, Wo)


# speedup vs baseline: 225.1415x; 1.0047x over previous
import functools

import jax
import jax.numpy as jnp
import numpy as np
from jax.experimental import pallas as pl
from jax.experimental.pallas import tpu as pltpu

_LANES = 128


def _roll_lanes(v, k):
    if k == 0:
        return v
    return jnp.concatenate([v[:, k:], v[:, :k]], axis=1)


def _conv_pool_kernel(x_ref, w1_ref, b_ref, hwm_ref, dm_ref, sc_ref,
                      pooled_ref, psum_ref, psq_ref, *, dims):
    g, cin, cout, h, w = dims
    hw = h * w
    bf16 = jnp.bfloat16

    mrows = []
    for e in range(g):
        x = x_ref[e].astype(bf16)

        slabs = []
        for kh in range(3):
            for kw in range(3):
                j = kh * 3 + kw
                sh = (kh - 1) * w + (kw - 1)
                slabs.append(_roll_lanes(x, sh) * hwm_ref[j:j + 1, :])
        u = jnp.concatenate(slabs, axis=0)

        t = jnp.dot(w1_ref[...], u,
                    preferred_element_type=jnp.float32)

        y = (_roll_lanes(t[:cout], -hw) * dm_ref[0:1, :]
             + t[cout:2 * cout]
             + _roll_lanes(t[2 * cout:], hw) * dm_ref[1:2, :])

        m = jnp.maximum(y + b_ref[...], 0.0)
        m = jnp.maximum(m, _roll_lanes(m, 1))
        m = jnp.maximum(m, _roll_lanes(m, w))
        m = jnp.maximum(m, _roll_lanes(m, hw))

        mrows.append(m.astype(bf16))

    mall = jnp.concatenate(mrows, axis=0)
    pall = jnp.dot(mall, sc_ref[...],
                   preferred_element_type=jnp.float32)

    psum = None
    psq = None
    for e in range(g):
        pc = pall[e * cout:(e + 1) * cout]
        pooled_ref[e] = pc.astype(bf16)
        s = jnp.sum(pc, axis=1, keepdims=True)
        sq = jnp.sum(pc * pc, axis=1, keepdims=True)
        psum = s if psum is None else psum + s
        psq = sq if psq is None else psq + sq

    psum_ref[...] = jnp.broadcast_to(psum, psum_ref.shape)
    psq_ref[...] = jnp.broadcast_to(psq, psq_ref.shape)


def _bn_kernel(pooled_ref, psum_ref, psq_ref, gamma_ref, beta_ref, o_ref,
               *, inv_count, eps):
    inv_rep = 1.0 / float(_LANES)
    s = jnp.sum(psum_ref[...], axis=1, keepdims=True) * inv_rep
    sq = jnp.sum(psq_ref[...], axis=1, keepdims=True) * inv_rep
    mean = s * inv_count
    var = jnp.maximum(sq * inv_count - mean * mean, 0.0)
    scale = jax.lax.rsqrt(var + eps) * gamma_ref[...]
    shift = beta_ref[...] - mean * scale
    o_ref[...] = pooled_ref[...].astype(jnp.float32) * scale + shift


def kernel(x, conv_w, conv_b, gamma, beta):
    eps = 1e-5
    B, Cin, D, H, W = x.shape
    Cout = conv_w.shape[0]
    Do, Ho, Wo = D // 2, H // 2, W // 2
    Nsp = Do * Ho * Wo
    S = D * H * W
    f32 = jnp.float32
    bf16 = jnp.bfloat16

    G = next(g for g in (12, 8, 6, 4, 2, 1) if B % g == 0)
    NT = B // G

    x_flat = x.reshape(B, Cin, S)

    w1 = (conv_w.astype(f32).transpose(2, 0, 3, 4, 1)
          .reshape(3 * Cout, 9 * Cin).astype(bf16))
    bias = conv_b.astype(f32).reshape(Cout, 1)

    li = np.arange(S)
    hh = (li // W) % H
    ww = li % W
    dd = li // (H * W)
    hwm = np.ones((9, S), np.float32)
    for kh in range(3):
        for kw in range(3):
            bad = np.zeros(S, bool)
            if kh == 0:
                bad |= hh == 0
            if kh == 2:
                bad |= hh == H - 1
            if kw == 0:
                bad |= ww == 0
            if kw == 2:
                bad |= ww == W - 1
            hwm[kh * 3 + kw, bad] = 0.0
    dm = np.ones((2, S), np.float32)
    dm[0, dd == 0] = 0.0
    dm[1, dd == D - 1] = 0.0

    sel = np.zeros((S, Nsp), np.float32)
    for od in range(Do):
        for oh in range(Ho):
            for ow in range(Wo):
                k = (2 * od * H + 2 * oh) * W + 2 * ow
                sel[k, (od * Ho + oh) * Wo + ow] = 1.0

    hwm_j = jnp.asarray(hwm, bf16)
    dm_j = jnp.asarray(dm, f32)
    sc_j = jnp.asarray(sel, bf16)

    conv_body = functools.partial(_conv_pool_kernel,
                                  dims=(G, Cin, Cout, H, W))
    pooled, psum, psq = pl.pallas_call(
        conv_body,
        out_shape=(
            jax.ShapeDtypeStruct((B, Cout, Nsp), bf16),
            jax.ShapeDtypeStruct((Cout, NT * _LANES), f32),
            jax.ShapeDtypeStruct((Cout, NT * _LANES), f32),
        ),
        grid=(NT,),
        in_specs=[
            pl.BlockSpec((G, Cin, S), lambda i: (i, 0, 0)),
            pl.BlockSpec((3 * Cout, 9 * Cin), lambda i: (0, 0)),
            pl.BlockSpec((Cout, 1), lambda i: (0, 0)),
            pl.BlockSpec((9, S), lambda i: (0, 0)),
            pl.BlockSpec((2, S), lambda i: (0, 0)),
            pl.BlockSpec((S, Nsp), lambda i: (0, 0)),
        ],
        out_specs=(
            pl.BlockSpec((G, Cout, Nsp), lambda i: (i, 0, 0)),
            pl.BlockSpec((Cout, _LANES), lambda i: (0, i)),
            pl.BlockSpec((Cout, _LANES), lambda i: (0, i)),
        ),
        compiler_params=pltpu.CompilerParams(
            dimension_semantics=("parallel",)),
    )(x_flat, w1, bias, hwm_j, dm_j, sc_j)

    bn_body = functools.partial(_bn_kernel,
                                inv_count=1.0 / float(B * Nsp), eps=float(eps))
    out_flat = pl.pallas_call(
        bn_body,
        out_shape=jax.ShapeDtypeStruct((B, Cout, Nsp), f32),
        grid=(NT,),
        in_specs=[
            pl.BlockSpec((G, Cout, Nsp), lambda i: (i, 0, 0)),
            pl.BlockSpec((Cout, NT * _LANES), lambda i: (0, 0)),
            pl.BlockSpec((Cout, NT * _LANES), lambda i: (0, 0)),
            pl.BlockSpec((Cout, 1), lambda i: (0, 0)),
            pl.BlockSpec((Cout, 1), lambda i: (0, 0)),
        ],
        out_specs=pl.BlockSpec((G, Cout, Nsp), lambda i: (i, 0, 0)),
        compiler_params=pltpu.CompilerParams(
            dimension_semantics=("parallel",)),
    )(pooled, psum, psq,
      gamma.astype(f32).reshape(Cout, 1), beta.astype(f32).reshape(Cout, 1))

    return out_flat.reshape(B, Cout, Do, Ho, Wo)


# speedup vs baseline: 237.8081x; 1.0563x over previous
import functools

import jax
import jax.numpy as jnp
import numpy as np
from jax.experimental import pallas as pl
from jax.experimental.pallas import tpu as pltpu

_LANES = 128


def _roll_lanes(v, k):
    if k == 0:
        return v
    return jnp.concatenate([v[:, k:], v[:, :k]], axis=1)


def _conv_pool_kernel(x_ref, w1_ref, b_ref, hwm_ref, dm_ref, sc_ref,
                      pooled_ref, psum_ref, psq_ref, *, dims):
    g, cin, cout, h, w = dims
    hw = h * w
    bf16 = jnp.bfloat16

    mrows = []
    for e in range(g):
        x = x_ref[e].astype(bf16)

        slabs = []
        for kh in range(3):
            for kw in range(3):
                j = kh * 3 + kw
                sh = (kh - 1) * w + (kw - 1)
                slabs.append(_roll_lanes(x, sh) * hwm_ref[j:j + 1, :])
        u = jnp.concatenate(slabs, axis=0)

        t = jnp.dot(w1_ref[...], u,
                    preferred_element_type=jnp.float32)

        y = (_roll_lanes(t[:cout], -hw) * dm_ref[0:1, :]
             + t[cout:2 * cout]
             + _roll_lanes(t[2 * cout:], hw) * dm_ref[1:2, :])

        m = jnp.maximum(y + b_ref[...], 0.0)
        m = jnp.maximum(m, _roll_lanes(m, 1))
        m = jnp.maximum(m, _roll_lanes(m, w))
        m = jnp.maximum(m, _roll_lanes(m, hw))

        mrows.append(m.astype(bf16))

    mall = jnp.concatenate(mrows, axis=0)
    pall = jnp.dot(mall, sc_ref[...],
                   preferred_element_type=jnp.float32)

    psum = None
    psq = None
    for e in range(g):
        pc = pall[e * cout:(e + 1) * cout]
        pooled_ref[e] = pc.astype(bf16)
        s = jnp.sum(pc, axis=1, keepdims=True)
        sq = jnp.sum(pc * pc, axis=1, keepdims=True)
        psum = s if psum is None else psum + s
        psq = sq if psq is None else psq + sq

    psum_ref[...] = jnp.broadcast_to(psum, psum_ref.shape)
    psq_ref[...] = jnp.broadcast_to(psq, psq_ref.shape)


def _bn_kernel(pooled_ref, psum_ref, psq_ref, gamma_ref, beta_ref, o_ref,
               *, inv_count, eps):
    inv_rep = 1.0 / float(_LANES)
    s = jnp.sum(psum_ref[...], axis=1, keepdims=True) * inv_rep
    sq = jnp.sum(psq_ref[...], axis=1, keepdims=True) * inv_rep
    mean = s * inv_count
    var = jnp.maximum(sq * inv_count - mean * mean, 0.0)
    scale = jax.lax.rsqrt(var + eps) * gamma_ref[...]
    shift = beta_ref[...] - mean * scale
    o_ref[...] = pooled_ref[...].astype(jnp.float32) * scale + shift


def kernel(x, conv_w, conv_b, gamma, beta):
    eps = 1e-5
    B, Cin, D, H, W = x.shape
    Cout = conv_w.shape[0]
    Do, Ho, Wo = D // 2, H // 2, W // 2
    Nsp = Do * Ho * Wo
    S = D * H * W
    f32 = jnp.float32
    bf16 = jnp.bfloat16

    G = next(g for g in (24, 12, 8, 6, 4, 2, 1) if B % g == 0)
    NT = B // G

    x_flat = x.reshape(B, Cin, S)

    w1 = (conv_w.astype(f32).transpose(2, 0, 3, 4, 1)
          .reshape(3 * Cout, 9 * Cin).astype(bf16))
    bias = conv_b.astype(f32).reshape(Cout, 1)

    li = np.arange(S)
    hh = (li // W) % H
    ww = li % W
    dd = li // (H * W)
    hwm = np.ones((9, S), np.float32)
    for kh in range(3):
        for kw in range(3):
            bad = np.zeros(S, bool)
            if kh == 0:
                bad |= hh == 0
            if kh == 2:
                bad |= hh == H - 1
            if kw == 0:
                bad |= ww == 0
            if kw == 2:
                bad |= ww == W - 1
            hwm[kh * 3 + kw, bad] = 0.0
    dm = np.ones((2, S), np.float32)
    dm[0, dd == 0] = 0.0
    dm[1, dd == D - 1] = 0.0

    sel = np.zeros((S, Nsp), np.float32)
    for od in range(Do):
        for oh in range(Ho):
            for ow in range(Wo):
                k = (2 * od * H + 2 * oh) * W + 2 * ow
                sel[k, (od * Ho + oh) * Wo + ow] = 1.0

    hwm_j = jnp.asarray(hwm, bf16)
    dm_j = jnp.asarray(dm, f32)
    sc_j = jnp.asarray(sel, bf16)

    conv_body = functools.partial(_conv_pool_kernel,
                                  dims=(G, Cin, Cout, H, W))
    pooled, psum, psq = pl.pallas_call(
        conv_body,
        out_shape=(
            jax.ShapeDtypeStruct((B, Cout, Nsp), bf16),
            jax.ShapeDtypeStruct((Cout, NT * _LANES), f32),
            jax.ShapeDtypeStruct((Cout, NT * _LANES), f32),
        ),
        grid=(NT,),
        in_specs=[
            pl.BlockSpec((G, Cin, S), lambda i: (i, 0, 0)),
            pl.BlockSpec((3 * Cout, 9 * Cin), lambda i: (0, 0)),
            pl.BlockSpec((Cout, 1), lambda i: (0, 0)),
            pl.BlockSpec((9, S), lambda i: (0, 0)),
            pl.BlockSpec((2, S), lambda i: (0, 0)),
            pl.BlockSpec((S, Nsp), lambda i: (0, 0)),
        ],
        out_specs=(
            pl.BlockSpec((G, Cout, Nsp), lambda i: (i, 0, 0)),
            pl.BlockSpec((Cout, _LANES), lambda i: (0, i)),
            pl.BlockSpec((Cout, _LANES), lambda i: (0, i)),
        ),
        compiler_params=pltpu.CompilerParams(
            dimension_semantics=("parallel",)),
    )(x_flat, w1, bias, hwm_j, dm_j, sc_j)

    bn_body = functools.partial(_bn_kernel,
                                inv_count=1.0 / float(B * Nsp), eps=float(eps))
    out_flat = pl.pallas_call(
        bn_body,
        out_shape=jax.ShapeDtypeStruct((B, Cout, Nsp), f32),
        grid=(NT,),
        in_specs=[
            pl.BlockSpec((G, Cout, Nsp), lambda i: (i, 0, 0)),
            pl.BlockSpec((Cout, NT * _LANES), lambda i: (0, 0)),
            pl.BlockSpec((Cout, NT * _LANES), lambda i: (0, 0)),
            pl.BlockSpec((Cout, 1), lambda i: (0, 0)),
            pl.BlockSpec((Cout, 1), lambda i: (0, 0)),
        ],
        out_specs=pl.BlockSpec((G, Cout, Nsp), lambda i: (i, 0, 0)),
        compiler_params=pltpu.CompilerParams(
            dimension_semantics=("parallel",)),
    )(pooled, psum, psq,
      gamma.astype(f32).reshape(Cout, 1), beta.astype(f32).reshape(Cout, 1))

    return out_flat.reshape(B, Cout, Do, Ho, Wo)


# speedup vs baseline: 254.8671x; 1.0717x over previous
import functools

import jax
import jax.numpy as jnp
import numpy as np
from jax.experimental import pallas as pl
from jax.experimental.pallas import tpu as pltpu

_LANES = 128


def _roll_lanes(v, k):
    if k == 0:
        return v
    return jnp.concatenate([v[:, k:], v[:, :k]], axis=1)


def _conv_pool_kernel(x_ref, w1_ref, b_ref, hwm_ref, dm_ref, sc_ref,
                      pooled_ref, psum_ref, psq_ref, *, dims):
    g, cin, cout, h, w = dims
    hw = h * w
    bf16 = jnp.bfloat16

    mrows = []
    for e in range(g):
        x = x_ref[e].astype(bf16)

        slabs = []
        for kh in range(3):
            for kw in range(3):
                j = kh * 3 + kw
                sh = (kh - 1) * w + (kw - 1)
                sl = _roll_lanes(x, sh)
                if j != 4:
                    sl = sl * hwm_ref[j:j + 1, :]
                slabs.append(sl)
        u = jnp.concatenate(slabs, axis=0)

        t = jnp.dot(w1_ref[...], u,
                    preferred_element_type=jnp.float32)

        y = (_roll_lanes(t[:cout], -hw) * dm_ref[0:1, :]
             + t[cout:2 * cout]
             + _roll_lanes(t[2 * cout:], hw) * dm_ref[1:2, :])

        m = jnp.maximum(y + b_ref[...], 0.0)
        m = m.astype(bf16)
        m = jnp.maximum(m, _roll_lanes(m, 1))
        m = jnp.maximum(m, _roll_lanes(m, w))
        m = jnp.maximum(m, _roll_lanes(m, hw))

        mrows.append(m)

    mall = jnp.concatenate(mrows, axis=0)
    pall = jnp.dot(mall, sc_ref[...],
                   preferred_element_type=jnp.float32)

    psum = None
    psq = None
    for e in range(g):
        pc = pall[e * cout:(e + 1) * cout]
        pooled_ref[e] = pc.astype(bf16)
        s = jnp.sum(pc, axis=1, keepdims=True)
        sq = jnp.sum(pc * pc, axis=1, keepdims=True)
        psum = s if psum is None else psum + s
        psq = sq if psq is None else psq + sq

    psum_ref[...] = jnp.broadcast_to(psum, psum_ref.shape)
    psq_ref[...] = jnp.broadcast_to(psq, psq_ref.shape)


def _bn_kernel(pooled_ref, psum_ref, psq_ref, gamma_ref, beta_ref, o_ref,
               *, inv_count, eps):
    inv_rep = 1.0 / float(_LANES)
    s = jnp.sum(psum_ref[...], axis=1, keepdims=True) * inv_rep
    sq = jnp.sum(psq_ref[...], axis=1, keepdims=True) * inv_rep
    mean = s * inv_count
    var = jnp.maximum(sq * inv_count - mean * mean, 0.0)
    scale = jax.lax.rsqrt(var + eps) * gamma_ref[...]
    shift = beta_ref[...] - mean * scale
    o_ref[...] = pooled_ref[...].astype(jnp.float32) * scale + shift


def kernel(x, conv_w, conv_b, gamma, beta):
    eps = 1e-5
    B, Cin, D, H, W = x.shape
    Cout = conv_w.shape[0]
    Do, Ho, Wo = D // 2, H // 2, W // 2
    Nsp = Do * Ho * Wo
    S = D * H * W
    f32 = jnp.float32
    bf16 = jnp.bfloat16

    G = next(g for g in (24, 12, 8, 6, 4, 2, 1) if B % g == 0)
    NT = B // G

    x_flat = x.reshape(B, Cin, S)

    w1 = (conv_w.astype(f32).transpose(2, 0, 3, 4, 1)
          .reshape(3 * Cout, 9 * Cin).astype(bf16))
    bias = conv_b.astype(f32).reshape(Cout, 1)

    li = np.arange(S)
    hh = (li // W) % H
    ww = li % W
    dd = li // (H * W)
    hwm = np.ones((9, S), np.float32)
    for kh in range(3):
        for kw in range(3):
            bad = np.zeros(S, bool)
            if kh == 0:
                bad |= hh == 0
            if kh == 2:
                bad |= hh == H - 1
            if kw == 0:
                bad |= ww == 0
            if kw == 2:
                bad |= ww == W - 1
            hwm[kh * 3 + kw, bad] = 0.0
    dm = np.ones((2, S), np.float32)
    dm[0, dd == 0] = 0.0
    dm[1, dd == D - 1] = 0.0

    sel = np.zeros((S, Nsp), np.float32)
    for od in range(Do):
        for oh in range(Ho):
            for ow in range(Wo):
                k = (2 * od * H + 2 * oh) * W + 2 * ow
                sel[k, (od * Ho + oh) * Wo + ow] = 1.0

    hwm_j = jnp.asarray(hwm, bf16)
    dm_j = jnp.asarray(dm, f32)
    sc_j = jnp.asarray(sel, bf16)

    conv_body = functools.partial(_conv_pool_kernel,
                                  dims=(G, Cin, Cout, H, W))
    pooled, psum, psq = pl.pallas_call(
        conv_body,
        out_shape=(
            jax.ShapeDtypeStruct((B, Cout, Nsp), bf16),
            jax.ShapeDtypeStruct((Cout, NT * _LANES), f32),
            jax.ShapeDtypeStruct((Cout, NT * _LANES), f32),
        ),
        grid=(NT,),
        in_specs=[
            pl.BlockSpec((G, Cin, S), lambda i: (i, 0, 0)),
            pl.BlockSpec((3 * Cout, 9 * Cin), lambda i: (0, 0)),
            pl.BlockSpec((Cout, 1), lambda i: (0, 0)),
            pl.BlockSpec((9, S), lambda i: (0, 0)),
            pl.BlockSpec((2, S), lambda i: (0, 0)),
            pl.BlockSpec((S, Nsp), lambda i: (0, 0)),
        ],
        out_specs=(
            pl.BlockSpec((G, Cout, Nsp), lambda i: (i, 0, 0)),
            pl.BlockSpec((Cout, _LANES), lambda i: (0, i)),
            pl.BlockSpec((Cout, _LANES), lambda i: (0, i)),
        ),
        compiler_params=pltpu.CompilerParams(
            dimension_semantics=("parallel",)),
    )(x_flat, w1, bias, hwm_j, dm_j, sc_j)

    bn_body = functools.partial(_bn_kernel,
                                inv_count=1.0 / float(B * Nsp), eps=float(eps))
    out_flat = pl.pallas_call(
        bn_body,
        out_shape=jax.ShapeDtypeStruct((B, Cout, Nsp), f32),
        grid=(NT,),
        in_specs=[
            pl.BlockSpec((G, Cout, Nsp), lambda i: (i, 0, 0)),
            pl.BlockSpec((Cout, NT * _LANES), lambda i: (0, 0)),
            pl.BlockSpec((Cout, NT * _LANES), lambda i: (0, 0)),
            pl.BlockSpec((Cout, 1), lambda i: (0, 0)),
            pl.BlockSpec((Cout, 1), lambda i: (0, 0)),
        ],
        out_specs=pl.BlockSpec((G, Cout, Nsp), lambda i: (i, 0, 0)),
        compiler_params=pltpu.CompilerParams(
            dimension_semantics=("parallel",)),
    )(pooled, psum, psq,
      gamma.astype(f32).reshape(Cout, 1), beta.astype(f32).reshape(Cout, 1))

    return out_flat.reshape(B, Cout, Do, Ho, Wo)


# speedup vs baseline: 273.3042x; 1.0723x over previous
import functools

import jax
import jax.numpy as jnp
import numpy as np
from jax.experimental import pallas as pl
from jax.experimental.pallas import tpu as pltpu

_LANES = 128


def _roll_lanes(v, k):
    if k == 0:
        return v
    return jnp.concatenate([v[:, k:], v[:, :k]], axis=1)


def _conv_pool_kernel(x_ref, w1_ref, b_ref, hwm_ref, dm_ref, sc_ref,
                      pooled_ref, psum_ref, psq_ref, *, dims):
    g, cin, cout, h, w = dims
    hw = h * w
    bf16 = jnp.bfloat16

    mrows = []
    for e in range(g):
        x = x_ref[e].astype(bf16)

        slabs = []
        for kh in range(3):
            for kw in range(3):
                j = kh * 3 + kw
                sh = (kh - 1) * w + (kw - 1)
                sl = _roll_lanes(x, sh)
                if j != 4:
                    sl = sl * hwm_ref[j:j + 1, :]
                slabs.append(sl)
        u = jnp.concatenate(slabs, axis=0)

        t = jnp.dot(w1_ref[...], u,
                    preferred_element_type=jnp.float32)

        y = (_roll_lanes(t[:cout], -hw) * dm_ref[0:1, :]
             + t[cout:2 * cout]
             + _roll_lanes(t[2 * cout:], hw) * dm_ref[1:2, :])

        m = jnp.maximum(y + b_ref[...], 0.0)
        m = m.astype(bf16)
        m = jnp.maximum(m, _roll_lanes(m, w))
        m = jnp.maximum(m, _roll_lanes(m, hw))

        mrows.append(m)

    mall = jnp.concatenate(mrows, axis=0)
    p2 = jnp.dot(mall, sc_ref[...],
                 preferred_element_type=jnp.float32)
    nsp = p2.shape[-1] // 2
    pall = jnp.maximum(p2[:, :nsp], p2[:, nsp:])

    psum = None
    psq = None
    for e in range(g):
        pc = pall[e * cout:(e + 1) * cout]
        pooled_ref[e] = pc.astype(bf16)
        s = jnp.sum(pc, axis=1, keepdims=True)
        sq = jnp.sum(pc * pc, axis=1, keepdims=True)
        psum = s if psum is None else psum + s
        psq = sq if psq is None else psq + sq

    psum_ref[...] = jnp.broadcast_to(psum, psum_ref.shape)
    psq_ref[...] = jnp.broadcast_to(psq, psq_ref.shape)


def _bn_kernel(pooled_ref, psum_ref, psq_ref, gamma_ref, beta_ref, o_ref,
               *, inv_count, eps):
    inv_rep = 1.0 / float(_LANES)
    s = jnp.sum(psum_ref[...], axis=1, keepdims=True) * inv_rep
    sq = jnp.sum(psq_ref[...], axis=1, keepdims=True) * inv_rep
    mean = s * inv_count
    var = jnp.maximum(sq * inv_count - mean * mean, 0.0)
    scale = jax.lax.rsqrt(var + eps) * gamma_ref[...]
    shift = beta_ref[...] - mean * scale
    o_ref[...] = pooled_ref[...].astype(jnp.float32) * scale + shift


def kernel(x, conv_w, conv_b, gamma, beta):
    eps = 1e-5
    B, Cin, D, H, W = x.shape
    Cout = conv_w.shape[0]
    Do, Ho, Wo = D // 2, H // 2, W // 2
    Nsp = Do * Ho * Wo
    S = D * H * W
    f32 = jnp.float32
    bf16 = jnp.bfloat16

    G = next(g for g in (24, 12, 8, 6, 4, 2, 1) if B % g == 0)
    NT = B // G

    x_flat = x.reshape(B, Cin, S)

    w1 = (conv_w.astype(f32).transpose(2, 0, 3, 4, 1)
          .reshape(3 * Cout, 9 * Cin).astype(bf16))
    bias = conv_b.astype(f32).reshape(Cout, 1)

    li = np.arange(S)
    hh = (li // W) % H
    ww = li % W
    dd = li // (H * W)
    hwm = np.ones((9, S), np.float32)
    for kh in range(3):
        for kw in range(3):
            bad = np.zeros(S, bool)
            if kh == 0:
                bad |= hh == 0
            if kh == 2:
                bad |= hh == H - 1
            if kw == 0:
                bad |= ww == 0
            if kw == 2:
                bad |= ww == W - 1
            hwm[kh * 3 + kw, bad] = 0.0
    dm = np.ones((2, S), np.float32)
    dm[0, dd == 0] = 0.0
    dm[1, dd == D - 1] = 0.0

    sel = np.zeros((S, 2 * Nsp), np.float32)
    for od in range(Do):
        for oh in range(Ho):
            for ow in range(Wo):
                k = (2 * od * H + 2 * oh) * W + 2 * ow
                q = (od * Ho + oh) * Wo + ow
                sel[k, q] = 1.0
                sel[k + 1, Nsp + q] = 1.0

    hwm_j = jnp.asarray(hwm, bf16)
    dm_j = jnp.asarray(dm, f32)
    sc_j = jnp.asarray(sel, bf16)

    conv_body = functools.partial(_conv_pool_kernel,
                                  dims=(G, Cin, Cout, H, W))
    pooled, psum, psq = pl.pallas_call(
        conv_body,
        out_shape=(
            jax.ShapeDtypeStruct((B, Cout, Nsp), bf16),
            jax.ShapeDtypeStruct((Cout, NT * _LANES), f32),
            jax.ShapeDtypeStruct((Cout, NT * _LANES), f32),
        ),
        grid=(NT,),
        in_specs=[
            pl.BlockSpec((G, Cin, S), lambda i: (i, 0, 0)),
            pl.BlockSpec((3 * Cout, 9 * Cin), lambda i: (0, 0)),
            pl.BlockSpec((Cout, 1), lambda i: (0, 0)),
            pl.BlockSpec((9, S), lambda i: (0, 0)),
            pl.BlockSpec((2, S), lambda i: (0, 0)),
            pl.BlockSpec((S, 2 * Nsp), lambda i: (0, 0)),
        ],
        out_specs=(
            pl.BlockSpec((G, Cout, Nsp), lambda i: (i, 0, 0)),
            pl.BlockSpec((Cout, _LANES), lambda i: (0, i)),
            pl.BlockSpec((Cout, _LANES), lambda i: (0, i)),
        ),
        compiler_params=pltpu.CompilerParams(
            dimension_semantics=("parallel",)),
    )(x_flat, w1, bias, hwm_j, dm_j, sc_j)

    bn_body = functools.partial(_bn_kernel,
                                inv_count=1.0 / float(B * Nsp), eps=float(eps))
    out_flat = pl.pallas_call(
        bn_body,
        out_shape=jax.ShapeDtypeStruct((B, Cout, Nsp), f32),
        grid=(NT,),
        in_specs=[
            pl.BlockSpec((G, Cout, Nsp), lambda i: (i, 0, 0)),
            pl.BlockSpec((Cout, NT * _LANES), lambda i: (0, 0)),
            pl.BlockSpec((Cout, NT * _LANES), lambda i: (0, 0)),
            pl.BlockSpec((Cout, 1), lambda i: (0, 0)),
            pl.BlockSpec((Cout, 1), lambda i: (0, 0)),
        ],
        out_specs=pl.BlockSpec((G, Cout, Nsp), lambda i: (i, 0, 0)),
        compiler_params=pltpu.CompilerParams(
            dimension_semantics=("parallel",)),
    )(pooled, psum, psq,
      gamma.astype(f32).reshape(Cout, 1), beta.astype(f32).reshape(Cout, 1))

    return out_flat.reshape(B, Cout, Do, Ho, Wo)


# speedup vs baseline: 306.9287x; 1.1230x over previous
import functools

import jax
import jax.numpy as jnp
import numpy as np
from jax.experimental import pallas as pl
from jax.experimental.pallas import tpu as pltpu

_LANES = 128


def _roll_lanes(v, k):
    if k == 0:
        return v
    return jnp.concatenate([v[:, k:], v[:, :k]], axis=1)


def _conv_pool_kernel(x_ref, w1_ref, b_ref, hwm_ref, dm_ref, sc_ref,
                      pooled_ref, psum_ref, psq_ref, *, dims):
    g, cin, cout, h, w = dims
    hw = h * w
    bf16 = jnp.bfloat16

    mrows = []
    for e in range(g):
        x = x_ref[e].astype(bf16)

        slabs = []
        for kh in range(3):
            for kw in range(3):
                j = kh * 3 + kw
                sh = (kh - 1) * w + (kw - 1)
                sl = _roll_lanes(x, sh)
                if j != 4:
                    sl = sl * hwm_ref[j:j + 1, :]
                slabs.append(sl)
        u = jnp.concatenate(slabs, axis=0)

        t = jnp.dot(w1_ref[...], u,
                    preferred_element_type=jnp.float32)

        y = (_roll_lanes(t[:cout], -hw) * dm_ref[0:1, :]
             + t[cout:2 * cout]
             + _roll_lanes(t[2 * cout:], hw) * dm_ref[1:2, :])

        m = jnp.maximum(y + b_ref[...], 0.0)
        m = m.astype(bf16)
        m = jnp.maximum(m, _roll_lanes(m, hw))
        m = jnp.concatenate(
            [m[:, (2 * od) * hw:(2 * od + 1) * hw]
             for od in range(m.shape[-1] // (2 * hw))], axis=1)
        m = jnp.maximum(m, _roll_lanes(m, w))

        mrows.append(m)

    mall = jnp.concatenate(mrows, axis=0)
    p2 = jnp.dot(mall, sc_ref[...],
                 preferred_element_type=jnp.float32)
    nsp = p2.shape[-1] // 2
    pall = jnp.maximum(p2[:, :nsp], p2[:, nsp:])

    psum = None
    psq = None
    for e in range(g):
        pc = pall[e * cout:(e + 1) * cout]
        pooled_ref[e] = pc.astype(bf16)
        s = jnp.sum(pc, axis=1, keepdims=True)
        sq = jnp.sum(pc * pc, axis=1, keepdims=True)
        psum = s if psum is None else psum + s
        psq = sq if psq is None else psq + sq

    psum_ref[...] = jnp.broadcast_to(psum, psum_ref.shape)
    psq_ref[...] = jnp.broadcast_to(psq, psq_ref.shape)


def _bn_kernel(pooled_ref, psum_ref, psq_ref, gamma_ref, beta_ref, o_ref,
               *, inv_count, eps):
    inv_rep = 1.0 / float(_LANES)
    s = jnp.sum(psum_ref[...], axis=1, keepdims=True) * inv_rep
    sq = jnp.sum(psq_ref[...], axis=1, keepdims=True) * inv_rep
    mean = s * inv_count
    var = jnp.maximum(sq * inv_count - mean * mean, 0.0)
    scale = jax.lax.rsqrt(var + eps) * gamma_ref[...]
    shift = beta_ref[...] - mean * scale
    o_ref[...] = pooled_ref[...].astype(jnp.float32) * scale + shift


def kernel(x, conv_w, conv_b, gamma, beta):
    eps = 1e-5
    B, Cin, D, H, W = x.shape
    Cout = conv_w.shape[0]
    Do, Ho, Wo = D // 2, H // 2, W // 2
    Nsp = Do * Ho * Wo
    S = D * H * W
    f32 = jnp.float32
    bf16 = jnp.bfloat16

    G = next(g for g in (24, 12, 8, 6, 4, 2, 1) if B % g == 0)
    NT = B // G

    x_flat = x.reshape(B, Cin, S)

    w1 = (conv_w.astype(f32).transpose(2, 0, 3, 4, 1)
          .reshape(3 * Cout, 9 * Cin).astype(bf16))
    bias = conv_b.astype(f32).reshape(Cout, 1)

    li = np.arange(S)
    hh = (li // W) % H
    ww = li % W
    dd = li // (H * W)
    hwm = np.ones((9, S), np.float32)
    for kh in range(3):
        for kw in range(3):
            bad = np.zeros(S, bool)
            if kh == 0:
                bad |= hh == 0
            if kh == 2:
                bad |= hh == H - 1
            if kw == 0:
                bad |= ww == 0
            if kw == 2:
                bad |= ww == W - 1
            hwm[kh * 3 + kw, bad] = 0.0
    dm = np.ones((2, S), np.float32)
    dm[0, dd == 0] = 0.0
    dm[1, dd == D - 1] = 0.0

    sel = np.zeros((S // 2, 2 * Nsp), np.float32)
    for od in range(Do):
        for oh in range(Ho):
            for ow in range(Wo):
                k = (od * H + 2 * oh) * W + 2 * ow
                q = (od * Ho + oh) * Wo + ow
                sel[k, q] = 1.0
                sel[k + 1, Nsp + q] = 1.0

    hwm_j = jnp.asarray(hwm, bf16)
    dm_j = jnp.asarray(dm, f32)
    sc_j = jnp.asarray(sel, bf16)

    conv_body = functools.partial(_conv_pool_kernel,
                                  dims=(G, Cin, Cout, H, W))
    pooled, psum, psq = pl.pallas_call(
        conv_body,
        out_shape=(
            jax.ShapeDtypeStruct((B, Cout, Nsp), bf16),
            jax.ShapeDtypeStruct((Cout, NT * _LANES), f32),
            jax.ShapeDtypeStruct((Cout, NT * _LANES), f32),
        ),
        grid=(NT,),
        in_specs=[
            pl.BlockSpec((G, Cin, S), lambda i: (i, 0, 0)),
            pl.BlockSpec((3 * Cout, 9 * Cin), lambda i: (0, 0)),
            pl.BlockSpec((Cout, 1), lambda i: (0, 0)),
            pl.BlockSpec((9, S), lambda i: (0, 0)),
            pl.BlockSpec((2, S), lambda i: (0, 0)),
            pl.BlockSpec((S // 2, 2 * Nsp), lambda i: (0, 0)),
        ],
        out_specs=(
            pl.BlockSpec((G, Cout, Nsp), lambda i: (i, 0, 0)),
            pl.BlockSpec((Cout, _LANES), lambda i: (0, i)),
            pl.BlockSpec((Cout, _LANES), lambda i: (0, i)),
        ),
        compiler_params=pltpu.CompilerParams(
            dimension_semantics=("parallel",)),
    )(x_flat, w1, bias, hwm_j, dm_j, sc_j)

    bn_body = functools.partial(_bn_kernel,
                                inv_count=1.0 / float(B * Nsp), eps=float(eps))
    out_flat = pl.pallas_call(
        bn_body,
        out_shape=jax.ShapeDtypeStruct((B, Cout, Nsp), f32),
        grid=(NT,),
        in_specs=[
            pl.BlockSpec((G, Cout, Nsp), lambda i: (i, 0, 0)),
            pl.BlockSpec((Cout, NT * _LANES), lambda i: (0, 0)),
            pl.BlockSpec((Cout, NT * _LANES), lambda i: (0, 0)),
            pl.BlockSpec((Cout, 1), lambda i: (0, 0)),
            pl.BlockSpec((Cout, 1), lambda i: (0, 0)),
        ],
        out_specs=pl.BlockSpec((G, Cout, Nsp), lambda i: (i, 0, 0)),
        compiler_params=pltpu.CompilerParams(
            dimension_semantics=("parallel",)),
    )(pooled, psum, psq,
      gamma.astype(f32).reshape(Cout, 1), beta.astype(f32).reshape(Cout, 1))

    return out_flat.reshape(B, Cout, Do, Ho, Wo)
